# baseline (speedup 1.0000x reference)
import functools

import jax
import jax.numpy as jnp
from jax import lax
from jax.experimental import pallas as pl
from jax.experimental.pallas import tpu as pltpu
from jax.experimental.pallas import tpu_sc as plsc

N = 10000
E = 320000
D_IN = 128
D_H = 64
N_GRAPHS = 64

NC = 2
NS = 16
NW = NC * NS
CH = 128
CHUNKS = E // CH
CPW = CHUNKS // NW
XTRA = CHUNKS - CPW * NW

_F32 = jnp.float32


def _dot(a, b, precision=lax.Precision.DEFAULT):
    return lax.dot_general(a, b, (((1,), (0,)), ((), ())),
                           preferred_element_type=_F32, precision=precision)


def _dot_t(a, b):
    return lax.dot_general(a, b, (((0,), (0,)), ((), ())),
                           preferred_element_type=_F32,
                           precision=lax.Precision.HIGHEST)


_ROWS_PER_SUB = 624
_TAIL = N - NS * _ROWS_PER_SUB


def _make_sc_agg(d_feat, nb):
    full_t = CPW // nb
    tailc = CPW - full_t * nb

    def body(feat_hbm, src_hbm, dst_hbm, zeros_hbm, out0_hbm, out1_hbm,
             sidx_v, didx_v, rows_v, acc_sh, gsem_arr, ssem_arr):
        gsem = [gsem_arr.at[j] for j in range(nb)]
        ssem = [ssem_arr.at[j] for j in range(nb)]
        cid = lax.axis_index("c")
        sid = lax.axis_index("s")
        wid = sid * NC + cid
        has_extra = wid < XTRA
        start = wid * CPW + lax.min(wid, XTRA)

        def _drain_scatter(j):
            pltpu.make_async_copy(feat_hbm.at[pl.ds(0, CH)],
                                  rows_v.at[j], ssem[j]).wait()

        def _gather(c, j):
            return pltpu.async_copy(feat_hbm.at[sidx_v.at[c]],
                                    rows_v.at[j], gsem[j])

        def _scatter(c, j):
            pltpu.async_copy(rows_v.at[j], acc_sh.at[didx_v.at[c]],
                             ssem[j], add=True)

        i1 = pltpu.async_copy(src_hbm.at[pl.ds(start, CPW)],
                              sidx_v.at[pl.ds(0, CPW)], gsem[0])
        i2 = pltpu.async_copy(dst_hbm.at[pl.ds(start, CPW)],
                              didx_v.at[pl.ds(0, CPW)], gsem[1])

        @pl.when(has_extra)
        def _():
            pltpu.sync_copy(src_hbm.at[pl.ds(start + CPW, 1)],
                            sidx_v.at[pl.ds(CPW, 1)])
            pltpu.sync_copy(dst_hbm.at[pl.ds(start + CPW, 1)],
                            didx_v.at[pl.ds(CPW, 1)])

        pltpu.sync_copy(zeros_hbm.at[pl.ds(sid * _ROWS_PER_SUB, _ROWS_PER_SUB)],
                        acc_sh.at[pl.ds(sid * _ROWS_PER_SUB, _ROWS_PER_SUB)])

        @pl.when(sid == NS - 1)
        def _():
            pltpu.sync_copy(zeros_hbm.at[pl.ds(NS * _ROWS_PER_SUB, _TAIL)],
                            acc_sh.at[pl.ds(NS * _ROWS_PER_SUB, _TAIL)])

        i1.wait()
        i2.wait()
        plsc.subcore_barrier()

        @pl.loop(0, full_t)
        def _(t):
            gathers = []
            for j in range(nb):
                @pl.when(t > 0)
                def _():
                    _drain_scatter(j)

                gathers.append(_gather(t * nb + j, j))
            for j in range(nb):
                gathers[j].wait()
                _scatter(t * nb + j, j)

        tail_g = []
        for j in range(tailc):
            _drain_scatter(j)
            tail_g.append(_gather(full_t * nb + j, j))
        for j in range(tailc):
            tail_g[j].wait()
            _scatter(full_t * nb + j, j)

        @pl.when(has_extra)
        def _():
            slot = tailc % nb
            _drain_scatter(slot)
            g = _gather(CPW, slot)
            g.wait()
            _scatter(CPW, slot)

        for j in range(nb):
            _drain_scatter(j)

        plsc.subcore_barrier()
        for core, out_hbm in ((0, out0_hbm), (1, out1_hbm)):
            @pl.when(cid == core)
            def _():
                pltpu.sync_copy(
                    acc_sh.at[pl.ds(sid * _ROWS_PER_SUB, _ROWS_PER_SUB)],
                    out_hbm.at[pl.ds(sid * _ROWS_PER_SUB, _ROWS_PER_SUB)])

            @pl.when(jnp.logical_and(cid == core, sid == NS - 1))
            def _():
                pltpu.sync_copy(acc_sh.at[pl.ds(NS * _ROWS_PER_SUB, _TAIL)],
                                out_hbm.at[pl.ds(NS * _ROWS_PER_SUB, _TAIL)])

    return pl.kernel(
        body,
        out_type=(jax.ShapeDtypeStruct((N, d_feat), _F32),
                  jax.ShapeDtypeStruct((N, d_feat), _F32)),
        mesh=plsc.VectorSubcoreMesh(core_axis_name="c", subcore_axis_name="s"),
        scratch_types=[
            pltpu.VMEM((CPW + 1, CH), jnp.int32),
            pltpu.VMEM((CPW + 1, CH), jnp.int32),
            pltpu.VMEM((nb, CH, d_feat), _F32),
            pltpu.VMEM_SHARED((N, d_feat), _F32),
            pltpu.SemaphoreType.DMA((nb,)),
            pltpu.SemaphoreType.DMA((nb,)),
        ],
        compiler_params=pltpu.CompilerParams(use_tc_tiling_on_sc=False),
    )


_sc_agg_64 = _make_sc_agg(D_H, 8)


def _bn_relu(z, gamma, beta):
    mu = jnp.mean(z, axis=0, keepdims=True)
    d = z - mu
    var = jnp.mean(d * d, axis=0, keepdims=True)
    return jnp.maximum(gamma * d * lax.rsqrt(var + 1e-5) + beta, 0.0)


def _tc_mid1_body(aggl0_ref, aggl1_ref, aggr0_ref, aggr1_ref, x_ref,
                  wrel_ref, wroot_ref, b_ref, gamma_ref, beta_ref, h_out):
    wrel = wrel_ref[...]
    z = (_dot(aggl0_ref[...] + aggl1_ref[...], wrel[:D_H])
         + _dot(aggr0_ref[...] + aggr1_ref[...], wrel[D_H:])
         + b_ref[...] + _dot(x_ref[...], wroot_ref[...]))
    h_out[...] = _bn_relu(z, gamma_ref[...], beta_ref[...])


def _tc_mid1(aggl0, aggl1, aggr0, aggr1, x, wrel, wroot, b, gamma, beta):
    return pl.pallas_call(
        _tc_mid1_body,
        out_shape=jax.ShapeDtypeStruct((N, D_H), _F32),
    )(aggl0, aggl1, aggr0, aggr1, x, wrel, wroot, b.reshape(1, D_H),
      gamma.reshape(1, D_H), beta.reshape(1, D_H))


def _tc_mid_body(agg0_ref, agg1_ref, x_ref, wrel_ref, wroot_ref, b_ref,
                 gamma_ref, beta_ref, h_out):
    z = (_dot(agg0_ref[...] + agg1_ref[...], wrel_ref[...]) + b_ref[...]
         + _dot(x_ref[...], wroot_ref[...]))
    h_out[...] = _bn_relu(z, gamma_ref[...], beta_ref[...])


def _tc_mid(agg0, agg1, x, wrel, wroot, b, gamma, beta):
    return pl.pallas_call(
        _tc_mid_body,
        out_shape=jax.ShapeDtypeStruct((N, D_H), _F32),
    )(agg0, agg1, x, wrel, wroot, b.reshape(1, D_H),
      gamma.reshape(1, D_H), beta.reshape(1, D_H))


def _tc_final_body(agg0_ref, agg1_ref, x_ref, wrel_ref, wroot_ref, b_ref,
                   batch_ref, wc1_ref, bc1_ref, wc2_ref, bc2_ref, out_ref):
    h = jnp.maximum(_dot(agg0_ref[...] + agg1_ref[...], wrel_ref[...])
                    + b_ref[...] + _dot(x_ref[...], wroot_ref[...]), 0.0)
    seg = batch_ref[...]
    ids = lax.broadcasted_iota(jnp.int32, (1, N_GRAPHS), 1)
    mask = (seg == ids).astype(_F32)
    s = _dot_t(mask, h)
    cnt = _dot_t(mask, jnp.ones((N, 1), _F32))
    g = s / jnp.maximum(cnt, 1.0)
    g = jnp.maximum(_dot(g, wc1_ref[...]) + bc1_ref[...], 0.0)
    out_ref[...] = _dot(g, wc2_ref[...]) + bc2_ref[...]


def _tc_final(agg0, agg1, x, wrel, wroot, b, batch, wc1, bc1, wc2, bc2):
    return pl.pallas_call(
        _tc_final_body,
        out_shape=jax.ShapeDtypeStruct((N_GRAPHS, 1), _F32),
    )(agg0, agg1, x, wrel, wroot, b.reshape(1, D_H), batch.reshape(N, 1),
      wc1, bc1.reshape(1, D_H), wc2, bc2.reshape(1, 1))


def kernel(x, edge_index, batch, W_rel1, b_rel1, W_root1, gamma1, beta1,
           W_rel2, b_rel2, W_root2, gamma2, beta2,
           W_rel3, b_rel3, W_root3, Wc1, bc1, Wc2, bc2):
    src = edge_index[0].reshape(CHUNKS, CH)
    dst = edge_index[1].reshape(CHUNKS, CH)
    src_l = (2 * edge_index[0]).reshape(CHUNKS, CH)
    src_r = (2 * edge_index[0] + 1).reshape(CHUNKS, CH)
    zeros64 = jnp.zeros((N, D_H), _F32)
    x2 = x.reshape(2 * N, D_H)

    al0, al1 = _sc_agg_64(x2, src_l, dst, zeros64)
    ar0, ar1 = _sc_agg_64(x2, src_r, dst, zeros64)
    h1 = _tc_mid1(al0, al1, ar0, ar1, x, W_rel1, W_root1, b_rel1,
                  gamma1, beta1)
    a0, a1 = _sc_agg_64(h1, src, dst, zeros64)
    h2 = _tc_mid(a0, a1, h1, W_rel2, W_root2, b_rel2, gamma2, beta2)
    a0, a1 = _sc_agg_64(h2, src, dst, zeros64)
    return _tc_final(a0, a1, h2, W_rel3, W_root3, b_rel3, batch,
                     Wc1, bc1, Wc2, bc2)
